# baseline (speedup 1.0000x reference)
import functools

import jax
import jax.numpy as jnp
from jax import lax
from jax.experimental import pallas as pl
from jax.experimental.pallas import tpu as pltpu
from jax.experimental.pallas import tpu_sc as plsc

B = 16
MAX_LEN = 2048
TOTAL = 16384
D = 512

TILE = 4096
N_TILES = TOTAL // TILE
PAD_ROWS = 5
P_PAD = PAD_ROWS * TILE

N_WORKERS = 32
OUT_ELEMS = B * MAX_LEN
CHUNK = OUT_ELEMS // N_WORKERS
WIN = CHUNK + 16


def _tc_body(cu_ref, x_ref, w1_ref, b1_ref, w2t_ref, b2_ref, out_ref, s_scr):
    i = pl.program_id(0)

    @pl.when(i < N_TILES)
    def _matmul():
        x_bf = x_ref[...].astype(jnp.bfloat16)
        w1_bf = w1_ref[...].astype(jnp.bfloat16)
        h = jnp.dot(x_bf, w1_bf, preferred_element_type=jnp.float32)
        h = jnp.maximum(h + b1_ref[...], 0.0)
        s = lax.dot_general(w2t_ref[...], h, (((1,), (1,)), ((), ())),
                            preferred_element_type=jnp.float32)
        s_scr[pl.ds(i, 1), :] = s + b2_ref[0, 0]

    @pl.when(i == N_TILES)
    def _softmax():
        s = s_scr[...]
        row = lax.broadcasted_iota(jnp.int32, (N_TILES, TILE), 0)
        col = lax.broadcasted_iota(jnp.int32, (N_TILES, TILE), 1)
        t = row * TILE + col
        neg_inf = jnp.float32(-jnp.inf)

        masks = []
        m_tok = jnp.zeros((N_TILES, TILE), jnp.float32)
        for b in range(B):
            mask = (t >= cu_ref[b]) & (t < cu_ref[b + 1])
            masks.append(mask)
            m_b = jnp.max(jnp.where(mask, s, neg_inf))
            m_tok = m_tok + jnp.where(mask, m_b, 0.0)

        e = jnp.exp(s - m_tok)
        d_tok = jnp.ones((N_TILES, TILE), jnp.float32)
        for b in range(B):
            sum_b = jnp.sum(jnp.where(masks[b], e, 0.0))
            d_tok = d_tok + jnp.where(masks[b], sum_b - 1.0, 0.0)

        out_ref[0:N_TILES, :] = e / d_tok
        out_ref[N_TILES:PAD_ROWS, :] = jnp.zeros(
            (PAD_ROWS - N_TILES, TILE), jnp.float32)


def _sc_gather_body(p_hbm, cu_hbm, out_hbm, cu_v, win_v, out_v):
    wid = lax.axis_index("s") * 2 + lax.axis_index("c")
    b = wid >> 1
    h = (wid & 1) * CHUNK
    pltpu.sync_copy(cu_hbm, cu_v)
    cu_vec = cu_v[pl.ds(b, 16)]
    seg_start = cu_vec[0]
    seg_end = cu_vec[1]
    start = seg_start + h
    ln = seg_end - start
    start8 = (start >> 3) << 3
    r = start - start8
    pltpu.sync_copy(p_hbm.at[pl.ds(pl.multiple_of(start8, 8), WIN)], win_v)

    def body(j, carry):
        col = j * 16 + lax.broadcasted_iota(jnp.int32, (16,), 0)
        v = win_v[pl.ds(r + j * 16, 16)]
        out_v[pl.ds(j * 16, 16)] = jnp.where(col < ln, v, 0.0)
        return carry

    lax.fori_loop(0, CHUNK // 16, body, 0)
    pltpu.sync_copy(out_v, out_hbm.at[pl.ds(wid * CHUNK, CHUNK)])


def kernel(flat, cu_seqlens, W1, b1, W2, b2):
    cu = cu_seqlens.astype(jnp.int32)
    b1r = b1.reshape(1, D)
    w2t = W2.reshape(1, D)
    b2r = b2.reshape(1, 1)

    probs = pl.pallas_call(
        _tc_body,
        grid_spec=pltpu.PrefetchScalarGridSpec(
            num_scalar_prefetch=1,
            grid=(N_TILES + 1,),
            in_specs=[
                pl.BlockSpec((TILE, D),
                             lambda i, c: (jnp.minimum(i, N_TILES - 1), 0)),
                pl.BlockSpec((D, D), lambda i, c: (0, 0)),
                pl.BlockSpec((1, D), lambda i, c: (0, 0)),
                pl.BlockSpec((1, D), lambda i, c: (0, 0)),
                pl.BlockSpec((1, 1), lambda i, c: (0, 0)),
            ],
            out_specs=pl.BlockSpec((PAD_ROWS, TILE), lambda i, c: (0, 0)),
            scratch_shapes=[pltpu.VMEM((N_TILES, TILE), jnp.float32)],
        ),
        out_shape=jax.ShapeDtypeStruct((PAD_ROWS, TILE), jnp.float32),
    )(cu, flat, W1, b1r, w2t, b2r)

    cu_pad = jnp.pad(cu, (0, 32 - (B + 1)))

    sc_gather = functools.partial(
        pl.kernel,
        mesh=plsc.VectorSubcoreMesh(core_axis_name="c", subcore_axis_name="s"),
        out_type=jax.ShapeDtypeStruct((OUT_ELEMS,), jnp.float32),
        scratch_types=[
            pltpu.VMEM((32,), jnp.int32),
            pltpu.VMEM((WIN,), jnp.float32),
            pltpu.VMEM((CHUNK,), jnp.float32),
        ],
        compiler_params=pltpu.CompilerParams(needs_layout_passes=False),
    )(_sc_gather_body)

    dense = sc_gather(probs.reshape(P_PAD), cu_pad)
    return dense.reshape(B, MAX_LEN)


# speedup vs baseline: 1.0467x; 1.0467x over previous
import functools

import jax
import jax.numpy as jnp
from jax import lax
from jax.experimental import pallas as pl
from jax.experimental.pallas import tpu as pltpu
from jax.experimental.pallas import tpu_sc as plsc

B = 16
MAX_LEN = 2048
TOTAL = 16384
D = 512

TILE = 2048
N_TILES = TOTAL // TILE
PAD_ROWS = 10
P_PAD = PAD_ROWS * TILE

N_WORKERS = 32
OUT_ELEMS = B * MAX_LEN
CHUNK = MAX_LEN
WIN = CHUNK + 16


def _tc_body(cu_ref, x_ref, w1_ref, b1_ref, w2t_ref, b2_ref, out_ref, s_scr):
    i = pl.program_id(0)

    @pl.when(i < N_TILES)
    def _matmul():
        x_bf = x_ref[...].astype(jnp.bfloat16)
        w1_bf = w1_ref[...].astype(jnp.bfloat16)
        h = jnp.dot(x_bf, w1_bf, preferred_element_type=jnp.float32)
        h = jnp.maximum(h + b1_ref[...], 0.0)
        s = lax.dot_general(w2t_ref[...], h, (((1,), (1,)), ((), ())),
                            preferred_element_type=jnp.float32)
        s_scr[pl.ds(i, 1), :] = s + b2_ref[0, 0]

    @pl.when(i == N_TILES)
    def _softmax():
        s = s_scr[...]
        row = lax.broadcasted_iota(jnp.int32, (N_TILES, TILE), 0)
        col = lax.broadcasted_iota(jnp.int32, (N_TILES, TILE), 1)
        t = row * TILE + col
        neg_inf = jnp.float32(-jnp.inf)

        masks = []
        m_tok = jnp.zeros((N_TILES, TILE), jnp.float32)
        for b in range(B):
            mask = (t >= cu_ref[b]) & (t < cu_ref[b + 1])
            masks.append(mask)
            m_b = jnp.max(jnp.where(mask, s, neg_inf))
            m_tok = m_tok + jnp.where(mask, m_b, 0.0)

        e = jnp.exp(s - m_tok)
        d_tok = jnp.ones((N_TILES, TILE), jnp.float32)
        for b in range(B):
            sum_b = jnp.sum(jnp.where(masks[b], e, 0.0))
            d_tok = d_tok + jnp.where(masks[b], sum_b - 1.0, 0.0)

        out_ref[0:N_TILES, :] = e / d_tok
        out_ref[N_TILES:PAD_ROWS, :] = jnp.zeros(
            (PAD_ROWS - N_TILES, TILE), jnp.float32)


def _sc_gather_body(p_hbm, cu_hbm, out_hbm, cu_v, win_v, out_v):
    wid = lax.axis_index("s")
    b = wid
    h = 0
    pltpu.sync_copy(cu_hbm, cu_v)
    cu_vec = cu_v[pl.ds(b, 16)]
    seg_start = cu_vec[0]
    seg_end = cu_vec[1]
    start = seg_start + h
    ln = seg_end - start
    start8 = (start >> 3) << 3
    r = start - start8
    pltpu.sync_copy(p_hbm.at[pl.ds(pl.multiple_of(start8, 8), WIN)], win_v)

    def body(j, carry):
        col = j * 16 + lax.broadcasted_iota(jnp.int32, (16,), 0)
        v = win_v[pl.ds(r + j * 16, 16)]
        out_v[pl.ds(j * 16, 16)] = jnp.where(col < ln, v, 0.0)
        return carry

    lax.fori_loop(0, CHUNK // 16, body, 0)
    pltpu.sync_copy(out_v, out_hbm.at[pl.ds(wid * CHUNK, CHUNK)])


def kernel(flat, cu_seqlens, W1, b1, W2, b2):
    cu = cu_seqlens.astype(jnp.int32)
    b1r = b1.reshape(1, D)
    w2t = W2.reshape(1, D)
    b2r = b2.reshape(1, 1)

    probs = pl.pallas_call(
        _tc_body,
        grid_spec=pltpu.PrefetchScalarGridSpec(
            num_scalar_prefetch=1,
            grid=(N_TILES + 1,),
            in_specs=[
                pl.BlockSpec((TILE, D),
                             lambda i, c: (jnp.minimum(i, N_TILES - 1), 0)),
                pl.BlockSpec((D, D), lambda i, c: (0, 0)),
                pl.BlockSpec((1, D), lambda i, c: (0, 0)),
                pl.BlockSpec((1, D), lambda i, c: (0, 0)),
                pl.BlockSpec((1, 1), lambda i, c: (0, 0)),
            ],
            out_specs=pl.BlockSpec((PAD_ROWS, TILE), lambda i, c: (0, 0)),
            scratch_shapes=[pltpu.VMEM((N_TILES, TILE), jnp.float32)],
        ),
        out_shape=jax.ShapeDtypeStruct((PAD_ROWS, TILE), jnp.float32),
    )(cu, flat, W1, b1r, w2t, b2r)

    cu_pad = jnp.pad(cu, (0, 32 - (B + 1)))

    sc_gather = functools.partial(
        pl.kernel,
        mesh=plsc.VectorSubcoreMesh(core_axis_name="c", subcore_axis_name="s", num_cores=1),
        out_type=jax.ShapeDtypeStruct((OUT_ELEMS,), jnp.float32),
        scratch_types=[
            pltpu.VMEM((32,), jnp.int32),
            pltpu.VMEM((WIN,), jnp.float32),
            pltpu.VMEM((CHUNK,), jnp.float32),
        ],
        compiler_params=pltpu.CompilerParams(needs_layout_passes=False),
    )(_sc_gather_body)

    dense = sc_gather(probs.reshape(P_PAD), cu_pad)
    return dense.reshape(B, MAX_LEN)
